# baseline (speedup 1.0000x reference)
import functools

import jax
import jax.numpy as jnp
from jax import lax
from jax.experimental import pallas as pl
from jax.experimental.pallas import tpu as pltpu
from jax.experimental.pallas import tpu_sc as plsc


def _mm_bias_body(x_ref, w_ref, b_ref, o_ref, *, relu):
    acc = jnp.dot(x_ref[...], w_ref[...], preferred_element_type=jnp.float32)
    acc = acc + b_ref[...]
    if relu:
        acc = jnp.maximum(acc, 0.0)
    o_ref[...] = acc


def _matmul_bias(x, w, b, relu, tn=512):
    n, k = x.shape
    m = w.shape[1]
    assert n % tn == 0
    return pl.pallas_call(
        functools.partial(_mm_bias_body, relu=relu),
        grid=(n // tn,),
        in_specs=[
            pl.BlockSpec((tn, k), lambda i: (i, 0)),
            pl.BlockSpec((k, m), lambda i: (0, 0)),
            pl.BlockSpec((1, m), lambda i: (0, 0)),
        ],
        out_specs=pl.BlockSpec((tn, m), lambda i: (i, 0)),
        out_shape=jax.ShapeDtypeStruct((n, m), jnp.float32),
    )(x, w, b.reshape(1, m))


def _conv2_vq_body(x_ref, w_ref, b_ref, cb_ref, lat_ref, idx_ref):
    z = jnp.dot(x_ref[...], w_ref[...], preferred_element_type=jnp.float32)
    z = z + b_ref[...]
    lat_ref[...] = z
    cb = cb_ref[...]
    zn = jnp.sum(z * z, axis=1, keepdims=True)
    cn = jnp.sum(cb * cb, axis=1)
    scores = zn - 2.0 * jnp.dot(z, cb.T, preferred_element_type=jnp.float32)
    scores = scores + cn[None, :]
    kk = scores.shape[1]
    mins = jnp.min(scores, axis=1, keepdims=True)
    kiota = lax.broadcasted_iota(jnp.int32, scores.shape, 1)
    idx = jnp.min(jnp.where(scores == mins, kiota, kk), axis=1)
    idx_ref[0, 0, :] = idx.astype(jnp.int32)


def _conv2_vq(x, w, b, codebook, tn=512):
    n, k = x.shape
    m = w.shape[1]
    kk = codebook.shape[0]
    assert n % tn == 0
    grid = n // tn
    lat, idx3 = pl.pallas_call(
        _conv2_vq_body,
        grid=(grid,),
        in_specs=[
            pl.BlockSpec((tn, k), lambda i: (i, 0)),
            pl.BlockSpec((k, m), lambda i: (0, 0)),
            pl.BlockSpec((1, m), lambda i: (0, 0)),
            pl.BlockSpec((kk, m), lambda i: (0, 0)),
        ],
        out_specs=[
            pl.BlockSpec((tn, m), lambda i: (i, 0)),
            pl.BlockSpec((1, 1, tn), lambda i: (i, 0, 0)),
        ],
        out_shape=[
            jax.ShapeDtypeStruct((n, m), jnp.float32),
            jax.ShapeDtypeStruct((grid, 1, tn), jnp.int32),
        ],
    )(x, w, b.reshape(1, m), codebook)
    return lat, idx3.reshape(n)


def _sc_gather(codebook, idx):
    b = idx.shape[0]
    d = 128
    codebook = jnp.pad(codebook, ((0, 0), (0, d - codebook.shape[1])))
    nc, ns = 2, 16
    nw = nc * ns
    bpw = b // nw
    assert b == bpw * nw and (bpw * nw) % 8 == 0
    mesh = plsc.VectorSubcoreMesh(core_axis_name="c", subcore_axis_name="s")

    @functools.partial(
        pl.kernel,
        mesh=mesh,
        out_type=jax.ShapeDtypeStruct((b, d), jnp.float32),
        scratch_types=[
            pltpu.VMEM((bpw,), jnp.int32),
            pltpu.VMEM((bpw, d), jnp.float32),
            pltpu.SemaphoreType.DMA,
        ],
    )
    def gather_k(cb_hbm, idx_hbm, out_hbm, idx_v, rows_v, sem):
        wid = lax.axis_index("s") * nc + lax.axis_index("c")
        base = wid * bpw
        pltpu.sync_copy(idx_hbm.at[pl.ds(base, bpw)], idx_v)
        pltpu.async_copy(cb_hbm.at[idx_v], rows_v, sem).wait()
        pltpu.sync_copy(rows_v, out_hbm.at[pl.ds(base, bpw)])

    return gather_k(codebook, idx)[:, :64]


def _im2col_s2(xp, oh, ow):
    taps = []
    for ky in range(4):
        for kx in range(4):
            taps.append(xp[:, ky:ky + 2 * (oh - 1) + 1:2,
                           kx:kx + 2 * (ow - 1) + 1:2, :])
    return jnp.concatenate(taps, axis=-1)


def _im2col9(xp, oh, ow):
    taps = [xp[:, ty:ty + oh, tx:tx + ow, :]
            for ty in range(3) for tx in range(3)]
    return jnp.concatenate(taps, axis=-1)


def _enc_weight(w):
    return jnp.transpose(w, (2, 3, 1, 0)).reshape(-1, w.shape[0])


def _dec_weight9(w):
    o, i = w.shape[0], w.shape[1]
    w9 = jnp.zeros((3, 3, i, 2, 2, o), jnp.float32)
    for py in range(2):
        for px in range(2):
            for dy in range(2):
                for dx in range(2):
                    w9 = w9.at[py + dy, px + dx, :, py, px, :].set(
                        jnp.transpose(w[:, :, 2 * dy + py, 2 * dx + px]))
    return w9.reshape(9 * i, 4 * o)


def _depth_to_space(y, b, h, w, o):
    y = y.reshape(b, h, w, 2, 2, o)
    return jnp.transpose(y, (0, 1, 3, 2, 4, 5)).reshape(b, 2 * h, 2 * w, o)


def _pad1(ximg):
    return jnp.pad(ximg, ((0, 0), (1, 1), (1, 1), (0, 0)))


def kernel(x, enc_w1, enc_b1, enc_w2, enc_b2, codebook,
           dec_w1, dec_b1, dec_w2, dec_b2):
    b = x.shape[0]
    d = enc_w1.shape[0]
    h1, w1 = x.shape[2] // 2, x.shape[3] // 2
    h2, w2 = h1 // 2, w1 // 2

    ximg = jnp.transpose(x, (0, 2, 3, 1))
    x1 = _im2col_s2(_pad1(ximg), h1, w1).reshape(b * h1 * w1, -1)
    h = _matmul_bias(x1, _enc_weight(enc_w1), enc_b1, relu=True)

    x2 = _im2col_s2(_pad1(h.reshape(b, h1, w1, d)), h2, w2)
    x2 = x2.reshape(b * h2 * w2, -1)
    latent_flat, idx_flat = _conv2_vq(x2, _enc_weight(enc_w2), enc_b2,
                                      codebook)

    q_flat = _sc_gather(codebook, idx_flat)

    x3 = _im2col9(_pad1(q_flat.reshape(b, h2, w2, d)), h2, w2)
    x3 = x3.reshape(b * h2 * w2, -1)
    g = _matmul_bias(x3, _dec_weight9(dec_w1), jnp.tile(dec_b1, 4), relu=True)
    gimg = _depth_to_space(g, b, h2, w2, d)

    x4 = _im2col9(_pad1(gimg), h1, w1).reshape(b * h1 * w1, -1)
    y = _matmul_bias(x4, _dec_weight9(dec_w2), jnp.tile(dec_b2, 4),
                     relu=False)
    x_hat = jnp.transpose(_depth_to_space(y, b, h1, w1, dec_w2.shape[0]),
                          (0, 3, 1, 2))

    latent = jnp.transpose(latent_flat.reshape(b, h2, w2, d), (0, 3, 1, 2))
    quantized = jnp.transpose(q_flat.reshape(b, h2, w2, d), (0, 3, 1, 2))
    return (x_hat, quantized, latent, idx_flat.reshape(b, h2, w2))


# speedup vs baseline: 2.2484x; 2.2484x over previous
import functools

import numpy as np
import jax
import jax.numpy as jnp
from jax import lax
from jax.experimental import pallas as pl
from jax.experimental.pallas import tpu as pltpu
from jax.experimental.pallas import tpu_sc as plsc


def _tap_conv_body(x_ref, w_ref, b_ref, o_ref, *, taps, oh, ow, relu):
    acc = None
    for t, (dy, dx) in enumerate(taps):
        xt = x_ref[0, dy:dy + oh, dx:dx + ow, :].reshape(oh * ow, -1)
        p = jnp.dot(xt, w_ref[t], preferred_element_type=jnp.float32)
        acc = p if acc is None else acc + p
    acc = acc + b_ref[...]
    if relu:
        acc = jnp.maximum(acc, 0.0)
    o_ref[0] = acc


def _tap_conv(ximg, w, b, taps, oh, ow, relu):
    bsz, hh, ww, cc = ximg.shape
    tt, _, m = w.shape
    return pl.pallas_call(
        functools.partial(_tap_conv_body, taps=taps, oh=oh, ow=ow, relu=relu),
        grid=(bsz,),
        in_specs=[
            pl.BlockSpec((1, hh, ww, cc), lambda i: (i, 0, 0, 0)),
            pl.BlockSpec((tt, cc, m), lambda i: (0, 0, 0)),
            pl.BlockSpec((1, m), lambda i: (0, 0)),
        ],
        out_specs=pl.BlockSpec((1, oh * ow, m), lambda i: (i, 0, 0)),
        out_shape=jax.ShapeDtypeStruct((bsz, oh * ow, m), jnp.float32),
    )(ximg, w, b.reshape(1, m))


def _conv2_vq_body(x_ref, w_ref, b_ref, cb_ref, lat_ref, idx_ref, *, oh, ow):
    acc = None
    for t, (dy, dx) in enumerate([(0, 0), (0, 1), (1, 0), (1, 1)]):
        xt = x_ref[0, dy:dy + oh, dx:dx + ow, :].reshape(oh * ow, -1)
        p = jnp.dot(xt, w_ref[t], preferred_element_type=jnp.float32)
        acc = p if acc is None else acc + p
    z = acc + b_ref[...]
    lat_ref[0] = z
    cb = cb_ref[...]
    zn = jnp.sum(z * z, axis=1, keepdims=True)
    cn = jnp.sum(cb * cb, axis=1)
    scores = zn - 2.0 * jnp.dot(z, cb.T, preferred_element_type=jnp.float32)
    scores = scores + cn[None, :]
    kk = scores.shape[1]
    mins = jnp.min(scores, axis=1, keepdims=True)
    kiota = lax.broadcasted_iota(jnp.int32, scores.shape, 1)
    idx = jnp.min(jnp.where(scores == mins, kiota, kk), axis=1)
    idx_ref[0, 0] = idx.astype(jnp.int32)


def _conv2_vq(ximg, w, b, codebook, oh, ow):
    bsz, hh, ww, cc = ximg.shape
    tt, _, m = w.shape
    kk = codebook.shape[0]
    n = oh * ow
    lat, idx = pl.pallas_call(
        functools.partial(_conv2_vq_body, oh=oh, ow=ow),
        grid=(bsz,),
        in_specs=[
            pl.BlockSpec((1, hh, ww, cc), lambda i: (i, 0, 0, 0)),
            pl.BlockSpec((tt, cc, m), lambda i: (0, 0, 0)),
            pl.BlockSpec((1, m), lambda i: (0, 0)),
            pl.BlockSpec((kk, m), lambda i: (0, 0)),
        ],
        out_specs=[
            pl.BlockSpec((1, n, m), lambda i: (i, 0, 0)),
            pl.BlockSpec((1, 1, n), lambda i: (i, 0, 0)),
        ],
        out_shape=[
            jax.ShapeDtypeStruct((bsz, n, m), jnp.float32),
            jax.ShapeDtypeStruct((bsz, 1, n), jnp.int32),
        ],
    )(ximg, w, b.reshape(1, m), codebook)
    return lat, idx.reshape(bsz * n)


def _sc_gather(codebook, idx):
    b = idx.shape[0]
    d = 128
    codebook = jnp.pad(codebook, ((0, 0), (0, d - codebook.shape[1])))
    nc, ns = 2, 16
    nw = nc * ns
    bpw = b // nw
    assert b == bpw * nw and (bpw * nw) % 8 == 0
    mesh = plsc.VectorSubcoreMesh(core_axis_name="c", subcore_axis_name="s")

    @functools.partial(
        pl.kernel,
        mesh=mesh,
        out_type=jax.ShapeDtypeStruct((b, d), jnp.float32),
        scratch_types=[
            pltpu.VMEM((bpw,), jnp.int32),
            pltpu.VMEM((bpw, d), jnp.float32),
            pltpu.SemaphoreType.DMA,
        ],
    )
    def gather_k(cb_hbm, idx_hbm, out_hbm, idx_v, rows_v, sem):
        wid = lax.axis_index("s") * nc + lax.axis_index("c")
        base = wid * bpw
        pltpu.sync_copy(idx_hbm.at[pl.ds(base, bpw)], idx_v)
        pltpu.async_copy(cb_hbm.at[idx_v], rows_v, sem).wait()
        pltpu.sync_copy(rows_v, out_hbm.at[pl.ds(base, bpw)])

    return gather_k(codebook, idx)[:, :64]


def _enc_tap_w(w):
    o, i = w.shape[0], w.shape[1]
    mats = []
    for dy in range(2):
        for dx in range(2):
            sub = w[:, :, 2 * dy:2 * dy + 2, 2 * dx:2 * dx + 2]
            mats.append(jnp.transpose(sub, (2, 3, 1, 0)).reshape(4 * i, o))
    return jnp.stack(mats)


def _placement(src_shape, dst_shape, entries):
    gidx = np.zeros(dst_shape, np.int32)
    mask = np.zeros(dst_shape, np.float32)
    for dst, src in entries:
        gidx[dst] = np.ravel_multi_index(src, src_shape)
        mask[dst] = 1.0
    return gidx, mask


def _dec1_entries(o, i):
    ent = []
    for ty in range(3):
        for tx in range(3):
            t = ty * 3 + tx
            for py in range(2):
                for px in range(2):
                    dy, dx = ty - py, tx - px
                    if dy in (0, 1) and dx in (0, 1):
                        for ci in range(i):
                            for co in range(o):
                                ent.append((
                                    (t, ci, (py * 2 + px) * o + co),
                                    (co, ci, 2 * dy + py, 2 * dx + px)))
    return ent


def _dec2_entries(o, i):
    ent = []
    for tj in range(3):
        for ti in range(3):
            t = tj * 3 + ti
            for ry in range(4):
                qy = ry % 2
                for dy in range(2):
                    pg = ry // 2 + qy + dy - 1 - 2 * (tj - 1)
                    if pg not in (0, 1):
                        continue
                    for rx in range(4):
                        qx = rx % 2
                        for dx in range(2):
                            pxg = rx // 2 + qx + dx - 1 - 2 * (ti - 1)
                            if pxg not in (0, 1):
                                continue
                            for ci in range(i):
                                for co in range(o):
                                    ent.append((
                                        (t, (pg * 2 + pxg) * i + ci,
                                         (ry * 4 + rx) * o + co),
                                        (co, ci, 2 * dy + qy, 2 * dx + qx)))
    return ent


@functools.lru_cache(maxsize=None)
def _dec_maps(o, i, which):
    ent = _dec1_entries(o, i) if which == 1 else _dec2_entries(o, i)
    if which == 1:
        dst = (9, i, 4 * o)
    else:
        dst = (9, 4 * i, 16 * o)
    return _placement((o, i, 4, 4), dst, ent)


def _dec_tap_w(w, which):
    o, i = int(w.shape[0]), int(w.shape[1])
    gidx, mask = _dec_maps(o, i, which)
    return jnp.take(w.reshape(-1), jnp.asarray(gidx)) * jnp.asarray(mask)


def _s2d(ximg):
    b, h2, w2, c = ximg.shape
    h, w = h2 // 2, w2 // 2
    y = ximg.reshape(b, h, 2, w, 2, c)
    return jnp.transpose(y, (0, 1, 3, 2, 4, 5)).reshape(b, h, w, 4 * c)


def _pad1(ximg):
    return jnp.pad(ximg, ((0, 0), (1, 1), (1, 1), (0, 0)))


_TAPS4 = [(0, 0), (0, 1), (1, 0), (1, 1)]
_TAPS9 = [(ty, tx) for ty in range(3) for tx in range(3)]


def kernel(x, enc_w1, enc_b1, enc_w2, enc_b2, codebook,
           dec_w1, dec_b1, dec_w2, dec_b2):
    b = x.shape[0]
    d = enc_w1.shape[0]
    h1 = x.shape[2] // 2
    h2 = h1 // 2
    co = dec_w2.shape[0]

    xs = _s2d(_pad1(jnp.transpose(x, (0, 2, 3, 1))))
    h = _tap_conv(xs, _enc_tap_w(enc_w1), enc_b1, _TAPS4, h1, h1, True)

    hs = _s2d(_pad1(h.reshape(b, h1, h1, d)))
    lat, idx_flat = _conv2_vq(hs, _enc_tap_w(enc_w2), enc_b2, codebook,
                              h2, h2)

    q_flat = _sc_gather(codebook, idx_flat)

    qp = _pad1(q_flat.reshape(b, h2, h2, d))
    gs = _tap_conv(qp, _dec_tap_w(dec_w1, 1), jnp.tile(dec_b1, 4),
                   _TAPS9, h2, h2, True)

    gsp = _pad1(gs.reshape(b, h2, h2, 4 * d))
    y = _tap_conv(gsp, _dec_tap_w(dec_w2, 2), jnp.tile(dec_b2, 16),
                  _TAPS9, h2, h2, False)

    x_hat = jnp.transpose(y.reshape(b, h2, h2, 4, 4, co),
                          (0, 5, 1, 3, 2, 4)).reshape(b, co, 4 * h2, 4 * h2)
    latent = jnp.transpose(lat.reshape(b, h2, h2, d), (0, 3, 1, 2))
    quantized = jnp.transpose(q_flat.reshape(b, h2, h2, d), (0, 3, 1, 2))
    return (x_hat, quantized, latent, idx_flat.reshape(b, h2, h2))
